# baseline (speedup 1.0000x reference)
import jax
import jax.numpy as jnp
from jax.experimental import pallas as pl
from jax.experimental.pallas import tpu as pltpu


def _deep_gcn_body(x_ref, w1_ref, b1_ref, w2_ref, b2_ref, w3_ref, b3_ref,
                   w4_ref, b4_ref, out_ref, a_ref):
    n = x_ref.shape[0]
    dinv = jax.lax.rsqrt(jnp.float32(n + 1))
    c = dinv * dinv

    g = jnp.dot(x_ref[...], w1_ref[...], preferred_element_type=jnp.float32)
    s = jnp.sum(g, axis=0, keepdims=True)
    a = jnp.maximum(g * c + (s * c + b1_ref[...]), 0.0)
    a_ref[:n, :] = a
    a_ref[n:, :] = jnp.zeros_like(a_ref[n:, :])
    a_ref[n:n + 1, :] = jnp.sum(a, axis=0, keepdims=True)

    for w_ref, b_ref in ((w2_ref, b2_ref), (w3_ref, b3_ref)):
        g = jnp.dot(a_ref[...], w_ref[...],
                    preferred_element_type=jnp.float32)
        s = g[n:n + 1, :]
        a = jnp.maximum(g[:n, :] * c + (s * c + b_ref[...]), 0.0)
        a_ref[:n, :] = a
        a_ref[n:n + 1, :] = jnp.sum(a, axis=0, keepdims=True)

    g = jnp.dot(a_ref[...], w4_ref[...], preferred_element_type=jnp.float32)
    s = g[n:n + 1, :]
    out_ref[...] = g[:n, :] * c + (s * c + b4_ref[...])


def kernel(x, W1, b1, W2, b2, W3, b3, W4, b4):
    n, _ = x.shape
    d_hid = W2.shape[0]
    d_out = W4.shape[1]
    n_pad = n + 8
    out = pl.pallas_call(
        _deep_gcn_body,
        out_shape=jax.ShapeDtypeStruct((n, d_out), jnp.float32),
        scratch_shapes=[pltpu.VMEM((n_pad, d_hid), jnp.float32)],
    )(x, W1, b1.reshape(1, -1), W2, b2.reshape(1, -1),
      W3, b3.reshape(1, -1), W4, b4.reshape(1, -1))
    return jnp.squeeze(out)


# speedup vs baseline: 1.0010x; 1.0010x over previous
import jax
import jax.numpy as jnp
from jax.experimental import pallas as pl
from jax.experimental.pallas import tpu as pltpu


def _deep_gcn_body(x_ref, w1_ref, b1_ref, w2_ref, b2_ref, w3_ref, b3_ref,
                   w4_ref, b4_ref, out_ref):
    n = x_ref.shape[0]
    dinv = jax.lax.rsqrt(jnp.float32(n + 1))
    c = dinv * dinv

    h = x_ref[...]
    layers = ((w1_ref, b1_ref, True), (w2_ref, b2_ref, True),
              (w3_ref, b3_ref, True), (w4_ref, b4_ref, False))
    for w_ref, b_ref, has_relu in layers:
        h = jnp.dot(h, w_ref[...], preferred_element_type=jnp.float32)
        s = jnp.sum(h, axis=0, keepdims=True)
        h = h * c + (s * c + b_ref[...])
        if has_relu:
            h = jnp.maximum(h, 0.0)
    out_ref[...] = h


def kernel(x, W1, b1, W2, b2, W3, b3, W4, b4):
    n, _ = x.shape
    d_out = W4.shape[1]
    out = pl.pallas_call(
        _deep_gcn_body,
        out_shape=jax.ShapeDtypeStruct((n, d_out), jnp.float32),
    )(x, W1, b1.reshape(1, -1), W2, b2.reshape(1, -1),
      W3, b3.reshape(1, -1), W4, b4.reshape(1, -1))
    return jnp.squeeze(out)
